# Initial kernel scaffold; baseline (speedup 1.0000x reference)
import jax
import jax.numpy as jnp
from jax.experimental import pallas as pl


def kernel(Q, K, V, Wq, bq, Wk, bk, Wv, bv, Wo, bo):
    raise NotImplementedError("write your pallas kernel here")


# speedup vs baseline: 3.1663x; 3.1663x over previous
import functools
import math

import jax
import jax.numpy as jnp
from jax.experimental import pallas as pl

N_HEADS = 16
D_MODEL = 1024
DK = D_MODEL // N_HEADS
L = 2048
BQ = 256


def _qkv_proj_kernel(x_ref, w_ref, b_ref, o_ref):
    x = x_ref[0]
    for h in range(N_HEADS):
        o_ref[0, h] = jnp.dot(x, w_ref[0, h], preferred_element_type=jnp.float32) + b_ref[0, h]


def _attn_kernel(q_ref, k_ref, v_ref, o_ref):
    q = q_ref[0, 0]
    k = k_ref[0, 0]
    v = v_ref[0, 0]
    s = jax.lax.dot_general(q, k, (((1,), (1,)), ((), ())),
                            preferred_element_type=jnp.float32)
    s = s * (1.0 / math.sqrt(DK))
    m = jnp.max(s, axis=1, keepdims=True)
    p = jnp.exp(s - m)
    l = jnp.sum(p, axis=1, keepdims=True)
    ctx = jnp.dot(p, v, preferred_element_type=jnp.float32)
    o_ref[0] = ctx / l


def _merge_kernel(c_ref, w_ref, b_ref, o_ref):
    acc = jnp.dot(c_ref[0], w_ref[0], preferred_element_type=jnp.float32)
    for h in range(1, N_HEADS):
        acc = acc + jnp.dot(c_ref[h], w_ref[h], preferred_element_type=jnp.float32)
    o_ref[...] = acc + b_ref[0][None, :]


@jax.jit
def _run(Q, K, V, Wq, bq, Wk, bk, Wv, bv, Wo, bo):
    B = Q.shape[0]
    H, dk = N_HEADS, DK

    X = jnp.stack([Q[0], K[0], V[0]])
    Ws = jnp.stack([Wq, Wk, Wv]).reshape(3, H, dk, D_MODEL).transpose(0, 1, 3, 2)
    bs = jnp.stack([bq, bk, bv]).reshape(3, H, 1, dk)

    nq = L // BQ
    P = pl.pallas_call(
        _qkv_proj_kernel,
        grid=(3, nq),
        in_specs=[
            pl.BlockSpec((1, BQ, D_MODEL), lambda m, i: (m, i, 0)),
            pl.BlockSpec((1, H, D_MODEL, dk), lambda m, i: (m, 0, 0, 0)),
            pl.BlockSpec((1, H, 1, dk), lambda m, i: (m, 0, 0, 0)),
        ],
        out_specs=pl.BlockSpec((1, H, BQ, dk), lambda m, i: (m, 0, i, 0)),
        out_shape=jax.ShapeDtypeStruct((3, H, L, dk), jnp.float32),
    )(X, Ws, bs)

    ctx = pl.pallas_call(
        _attn_kernel,
        grid=(H, nq),
        in_specs=[
            pl.BlockSpec((1, 1, BQ, dk), lambda h, i: (0, h, i, 0)),
            pl.BlockSpec((1, 1, L, dk), lambda h, i: (1, h, 0, 0)),
            pl.BlockSpec((1, 1, L, dk), lambda h, i: (2, h, 0, 0)),
        ],
        out_specs=pl.BlockSpec((1, BQ, dk), lambda h, i: (h, i, 0)),
        out_shape=jax.ShapeDtypeStruct((H, L, dk), jnp.float32),
    )(P, P, P)

    WoH = Wo.T.reshape(H, dk, D_MODEL)
    out = pl.pallas_call(
        _merge_kernel,
        grid=(nq,),
        in_specs=[
            pl.BlockSpec((H, BQ, dk), lambda i: (0, i, 0)),
            pl.BlockSpec((H, dk, D_MODEL), lambda i: (0, 0, 0)),
            pl.BlockSpec((1, D_MODEL), lambda i: (0, 0)),
        ],
        out_specs=pl.BlockSpec((BQ, D_MODEL), lambda i: (i, 0)),
        out_shape=jax.ShapeDtypeStruct((L, D_MODEL), jnp.float32),
    )(ctx, WoH, bo.reshape(1, D_MODEL))

    return out.reshape(B, L, D_MODEL)


def kernel(Q, K, V, Wq, bq, Wk, bk, Wv, bv, Wo, bo):
    return _run(Q, K, V, Wq, bq, Wk, bk, Wv, bv, Wo, bo)


# speedup vs baseline: 4.3878x; 1.3858x over previous
import functools
import math

import jax
import jax.numpy as jnp
from jax.experimental import pallas as pl

N_HEADS = 16
D_MODEL = 1024
DK = D_MODEL // N_HEADS
L = 2048
BQ = 256
LOG2E = 1.4426950408889634


def _qkv_proj_kernel(x_ref, w_ref, b_ref, o_ref):
    x = x_ref[0]
    for h in range(N_HEADS):
        acc = jnp.dot(x, w_ref[0, h], preferred_element_type=jnp.float32)
        o_ref[0, h] = (acc + b_ref[0, h]).astype(jnp.bfloat16)


def _attn_kernel(q_ref, k_ref, v_ref, o_ref):
    scale = LOG2E / math.sqrt(DK)
    ctxs = []
    for h in range(2):
        q = (q_ref[0, h].astype(jnp.float32) * scale).astype(jnp.bfloat16)
        s = jax.lax.dot_general(q, k_ref[0, h], (((1,), (1,)), ((), ())),
                                preferred_element_type=jnp.float32)
        m = jnp.max(s, axis=1, keepdims=True)
        p = jnp.exp2(s - m)
        l = jnp.sum(p, axis=1, keepdims=True)
        ctx = jnp.dot(p.astype(jnp.bfloat16), v_ref[0, h],
                      preferred_element_type=jnp.float32)
        ctxs.append(ctx / l)
    o_ref[...] = jnp.concatenate(ctxs, axis=1).astype(jnp.bfloat16)


def _out_proj_kernel(c_ref, w_ref, b_ref, o_ref):
    acc = jnp.dot(c_ref[...], w_ref[...], preferred_element_type=jnp.float32)
    o_ref[...] = acc + b_ref[0][None, :]


@jax.jit
def _run(Q, K, V, Wq, bq, Wk, bk, Wv, bv, Wo, bo):
    B = Q.shape[0]
    H, dk = N_HEADS, DK

    X = jnp.stack([Q[0], K[0], V[0]]).astype(jnp.bfloat16)
    Ws = (jnp.stack([Wq, Wk, Wv]).reshape(3, H, dk, D_MODEL)
          .transpose(0, 1, 3, 2).astype(jnp.bfloat16))
    bs = jnp.stack([bq, bk, bv]).reshape(3, H, 1, dk)

    nq = L // BQ
    P = pl.pallas_call(
        _qkv_proj_kernel,
        grid=(3, nq),
        in_specs=[
            pl.BlockSpec((1, BQ, D_MODEL), lambda m, i: (m, i, 0)),
            pl.BlockSpec((1, H, D_MODEL, dk), lambda m, i: (m, 0, 0, 0)),
            pl.BlockSpec((1, H, 1, dk), lambda m, i: (m, 0, 0, 0)),
        ],
        out_specs=pl.BlockSpec((1, H, BQ, dk), lambda m, i: (m, 0, i, 0)),
        out_shape=jax.ShapeDtypeStruct((3, H, L, dk), jnp.bfloat16),
    )(X, Ws, bs)

    ctx = pl.pallas_call(
        _attn_kernel,
        grid=(H // 2, nq),
        in_specs=[
            pl.BlockSpec((1, 2, BQ, dk), lambda h, i: (0, h, i, 0)),
            pl.BlockSpec((1, 2, L, dk), lambda h, i: (1, h, 0, 0)),
            pl.BlockSpec((1, 2, L, dk), lambda h, i: (2, h, 0, 0)),
        ],
        out_specs=pl.BlockSpec((BQ, 2 * dk), lambda h, i: (i, h)),
        out_shape=jax.ShapeDtypeStruct((L, D_MODEL), jnp.bfloat16),
    )(P, P, P)

    out = pl.pallas_call(
        _out_proj_kernel,
        grid=(nq,),
        in_specs=[
            pl.BlockSpec((BQ, D_MODEL), lambda i: (i, 0)),
            pl.BlockSpec((D_MODEL, D_MODEL), lambda i: (0, 0)),
            pl.BlockSpec((1, D_MODEL), lambda i: (0, 0)),
        ],
        out_specs=pl.BlockSpec((BQ, D_MODEL), lambda i: (i, 0)),
        out_shape=jax.ShapeDtypeStruct((L, D_MODEL), jnp.float32),
    )(ctx, Wo.T.astype(jnp.bfloat16), bo.reshape(1, D_MODEL))

    return out.reshape(B, L, D_MODEL)


def kernel(Q, K, V, Wq, bq, Wk, bk, Wv, bv, Wo, bo):
    return _run(Q, K, V, Wq, bq, Wk, bk, Wv, bv, Wo, bo)
